# baseline (speedup 1.0000x reference)
import jax
import jax.numpy as jnp
from jax import lax
from jax.experimental import pallas as pl
from jax.experimental.pallas import tpu as pltpu
from jax.experimental.pallas import tpu_sc as plsc

_NUM_CLASSES = 90
_C_PAD = 96
_FEA = 128
_BATCH = 16384
_MARGIN = 1.0
_NEG_BIG = -1e30

_NC, _NS = 2, 16
_NW = _NC * _NS
_ROWS_PER_W = _BATCH // _NW
_GROUPS = _ROWS_PER_W // 16

_B_BLK = 2048
_W_PER_BLK = _B_BLK // _ROWS_PER_W


def _tc_dots_kernel(x_ref, c_ref, out_ref):
    c = c_ref[...]
    nrm = jnp.sqrt(jnp.sum(c * c, axis=1, keepdims=True))
    nc = c / (nrm + 1e-12)
    d = lax.dot_general(nc, x_ref[...], (((1,), (1,)), ((), ())),
                        preferred_element_type=jnp.float32)
    row = lax.broadcasted_iota(jnp.int32, d.shape, 0)
    d = jnp.where(row < _NUM_CLASSES, d, _NEG_BIG)
    for w in range(_W_PER_BLK):
        out_ref[w] = d[:, w * _ROWS_PER_W:(w + 1) * _ROWS_PER_W]


def _sc_loss_kernel(dots_hbm, lab_hbm, out_hbm, dots_v, lab_v, acc_v):
    wid = lax.axis_index("s") * _NC + lax.axis_index("c")
    pltpu.sync_copy(dots_hbm.at[wid], dots_v)
    pltpu.sync_copy(lab_hbm.at[pl.ds(wid * _ROWS_PER_W, _ROWS_PER_W)], lab_v)

    def body(g, acc):
        off = g * 16
        labv = lab_v[pl.ds(off, 16)]
        neg = jnp.full((16,), _NEG_BIG, jnp.float32)
        m = [neg] * 8
        p = [neg] * 4
        for j in range(_C_PAD):
            v = dots_v[j, pl.ds(off, 16)]
            own = labv == j
            m[j % 8] = jnp.maximum(m[j % 8], jnp.where(own, _NEG_BIG, v))
            p[j % 4] = jnp.maximum(p[j % 4], jnp.where(own, v, _NEG_BIG))
        m0 = jnp.maximum(jnp.maximum(m[0], m[1]), jnp.maximum(m[2], m[3]))
        m1 = jnp.maximum(jnp.maximum(m[4], m[5]), jnp.maximum(m[6], m[7]))
        mx = jnp.maximum(m0, m1)
        px = jnp.maximum(jnp.maximum(p[0], p[1]), jnp.maximum(p[2], p[3]))
        return acc + jnp.maximum(_MARGIN + mx - px, 0.0)

    acc = lax.fori_loop(0, _GROUPS, body, jnp.zeros((16,), jnp.float32))
    acc_v[...] = acc
    pltpu.sync_copy(acc_v, out_hbm.at[wid])


def kernel(x, labels, centers):
    labels = labels.astype(jnp.int32)
    cpad = jnp.pad(centers, ((0, _C_PAD - _NUM_CLASSES), (0, 0)))

    dots = pl.pallas_call(
        _tc_dots_kernel,
        grid=(_BATCH // _B_BLK,),
        in_specs=[
            pl.BlockSpec((_B_BLK, _FEA), lambda i: (i, 0)),
            pl.BlockSpec((_C_PAD, _FEA), lambda i: (0, 0)),
        ],
        out_specs=pl.BlockSpec((_W_PER_BLK, _C_PAD, _ROWS_PER_W),
                               lambda i: (i, 0, 0)),
        out_shape=jax.ShapeDtypeStruct((_NW, _C_PAD, _ROWS_PER_W),
                                       jnp.float32),
    )(x, cpad)

    partials = pl.kernel(
        _sc_loss_kernel,
        out_type=jax.ShapeDtypeStruct((_NW, 16), jnp.float32),
        mesh=plsc.VectorSubcoreMesh(core_axis_name="c", subcore_axis_name="s"),
        scratch_types=[
            pltpu.VMEM((_C_PAD, _ROWS_PER_W), jnp.float32),
            pltpu.VMEM((_ROWS_PER_W,), jnp.int32),
            pltpu.VMEM((16,), jnp.float32),
        ],
    )(dots, labels)

    return jnp.sum(partials) / _BATCH
